# baseline (speedup 1.0000x reference)
import dataclasses
import functools

import jax
import jax.numpy as jnp
from jax import lax
from jax.experimental import pallas as pl
from jax.experimental.pallas import tpu as pltpu
from jax.experimental.pallas import tpu_sc as plsc

N = 10000
NP = 10240
E = 320000
EP = 327680
H = 128
G = 64
VOCAB = 28
RB = 1024
NBLK = NP // RB
NTILES = 32
E_TILE = EP // NTILES
CH = 80
N_CH = E_TILE // CH
DCH = 1024
N_DCH = E_TILE // DCH
ROWS_PER_TILE = NP // 16

_F32 = jnp.float32
_HIGH = lax.Precision.HIGHEST

_mesh = plsc.VectorSubcoreMesh(core_axis_name="c", subcore_axis_name="s",
                               num_cores=2, num_subcores=16)

_sc_params = pltpu.CompilerParams()
if "needs_layout_passes" in pltpu.CompilerParams.__dataclass_fields__:
    _sc_params = dataclasses.replace(_sc_params, needs_layout_passes=False)


@functools.partial(
    pl.kernel,
    out_type=jax.ShapeDtypeStruct((NTILES, NP), _F32),
    mesh=_mesh,
    scratch_types=[
        pltpu.VMEM((NP,), _F32),
        pltpu.VMEM((DCH,), jnp.int32),
        pltpu.VMEM((DCH,), _F32),
    ],
    compiler_params=_sc_params,
)
def _deg_kernel(col_hbm, ew_hbm, out_hbm, deg_ref, col_ref, ew_ref):
    c = lax.axis_index("c")
    s = lax.axis_index("s")
    tile = s * 2 + c
    zeros16 = jnp.zeros((16,), _F32)

    @pl.loop(0, NP // 16)
    def _(i):
        deg_ref[pl.ds(i * 16, 16)] = zeros16

    base = tile * E_TILE

    @pl.loop(0, N_DCH)
    def _(k):
        off = base + k * DCH
        pltpu.sync_copy(col_hbm.at[pl.ds(off, DCH)], col_ref)
        pltpu.sync_copy(ew_hbm.at[pl.ds(off, DCH)], ew_ref)

        @pl.loop(0, DCH // 16)
        def _(g):
            c16 = col_ref[pl.ds(g * 16, 16)]
            e16 = ew_ref[pl.ds(g * 16, 16)]
            plsc.addupdate_scatter(deg_ref, [c16], e16)

    pltpu.sync_copy(deg_ref, out_hbm.at[tile])


NB = 4
TOT_CH = EP // CH
NCH_C0 = 128
NCH_C1 = 128
assert NCH_C0 % NB == 0 and NCH_C1 % NB == 0
assert 16 * (NCH_C0 + NCH_C1) == TOT_CH


def _edge_pipeline(base, nch, hws_hbm, row_hbm, cw_hbm,
                   ROW, CW, M, SR, SCW, SG, SS, acc_ref):
    for b in range(min(NB, nch)):
        pltpu.async_copy(row_hbm.at[base + b], ROW[b], SR[b])
    for b in range(min(2, nch)):
        pltpu.async_copy(cw_hbm.at[base + b], CW[b], SCW[b])
    for b in range(min(2, nch)):
        pltpu.make_async_copy(row_hbm.at[base + b], ROW[b], SR[b]).wait()
        pltpu.async_copy(hws_hbm.at[ROW[b]], M[b], SG[b])

    @pl.loop(0, nch // NB)
    def _(kk):
        for b in range(NB):
            k = kk * NB + b
            b2 = (b + 2) % NB

            @pl.when(k >= 2)
            def _():
                pltpu.make_async_copy(
                    M[b2], acc_ref.at[CW[b2].at[1]], SS[b2]).wait()

            @pl.when(k + 2 < nch)
            def _():
                pltpu.async_copy(cw_hbm.at[base + k + 2], CW[b2], SCW[b2])
                pltpu.make_async_copy(
                    row_hbm.at[base + k + 2], ROW[b2], SR[b2]).wait()
                pltpu.async_copy(hws_hbm.at[ROW[b2]], M[b2], SG[b2])

            pltpu.make_async_copy(
                hws_hbm.at[ROW[b]], M[b], SG[b]).wait()

            @pl.when(k + 4 < nch)
            def _():
                pltpu.async_copy(row_hbm.at[base + k + 4], ROW[b], SR[b])

            pltpu.make_async_copy(cw_hbm.at[base + k], CW[b], SCW[b]).wait()

            @pl.loop(0, CH // 16)
            def _(g):
                e16 = plsc.bitcast(CW[b][2, pl.ds(g * 16, 16)], _F32)
                for j in range(16):
                    bj = jnp.broadcast_to(e16[j], (16,))
                    for f in range(8):
                        msl = (g * 16 + j, pl.ds(f * 16, 16))
                        M[b][msl] = M[b][msl] * bj

            pltpu.async_copy(M[b], acc_ref.at[CW[b].at[1]], SS[b], add=True)

    if nch >= 2:
        pltpu.make_async_copy(
            M[(nch - 2) % NB], acc_ref.at[CW[(nch - 2) % NB].at[1]],
            SS[(nch - 2) % NB]).wait()
    if nch >= 1:
        pltpu.make_async_copy(
            M[(nch - 1) % NB], acc_ref.at[CW[(nch - 1) % NB].at[1]],
            SS[(nch - 1) % NB]).wait()


@functools.partial(
    pl.kernel,
    out_type=jax.ShapeDtypeStruct((2, NP, H), _F32),
    mesh=_mesh,
    scratch_types=(
        [pltpu.VMEM((CH,), jnp.int32)] * NB
        + [pltpu.VMEM((3, CH), jnp.int32)] * NB
        + [pltpu.VMEM((CH, H), _F32)] * NB
        + [pltpu.VMEM_SHARED((NP, H), _F32)]
        + [pltpu.SemaphoreType.DMA] * (4 * NB)
    ),
    compiler_params=_sc_params,
)
def _msg_kernel(hws_hbm, row_hbm, cw_hbm, zeros_hbm, out_hbm,
                r0, r1, r2, r3, w0, w1, w2, w3, m0, m1, m2, m3, acc_ref,
                *sems):
    c = lax.axis_index("c")
    s = lax.axis_index("s")
    ROW = (r0, r1, r2, r3)
    CW = (w0, w1, w2, w3)
    M = (m0, m1, m2, m3)
    SR = sems[0:4]
    SCW = sems[4:8]
    SG = sems[8:12]
    SS = sems[12:16]
    pltpu.sync_copy(zeros_hbm.at[pl.ds(s * ROWS_PER_TILE, ROWS_PER_TILE)],
                    acc_ref.at[pl.ds(s * ROWS_PER_TILE, ROWS_PER_TILE)])
    plsc.subcore_barrier()

    if NCH_C0 > 0:
        @pl.when(c == 0)
        def _():
            _edge_pipeline(s * NCH_C0, NCH_C0, hws_hbm, row_hbm, cw_hbm,
                           ROW, CW, M, SR, SCW, SG, SS, acc_ref)
    if NCH_C1 > 0:
        @pl.when(c == 1)
        def _():
            _edge_pipeline(16 * NCH_C0 + s * NCH_C1, NCH_C1,
                           hws_hbm, row_hbm, cw_hbm,
                           ROW, CW, M, SR, SCW, SG, SS, acc_ref)

    plsc.subcore_barrier()
    pltpu.sync_copy(acc_ref.at[pl.ds(s * ROWS_PER_TILE, ROWS_PER_TILE)],
                    out_hbm.at[c].at[pl.ds(s * ROWS_PER_TILE, ROWS_PER_TILE)])


def _dot(a, b):
    return lax.dot_general(a, b, (((1,), (0,)), ((), ())),
                           precision=_HIGH, preferred_element_type=_F32)


def _prologue_body(xb_ref, pe_ref, emb_ref, wt_ref, btr_ref, w1_ref, disc_ref,
                   h_ref, hws1_ref):
    xb = xb_ref[...]
    iot = lax.broadcasted_iota(jnp.int32, (RB, 32), 1)
    oh = (xb == iot).astype(_F32)
    h = _dot(oh, emb_ref[...]) + _dot(pe_ref[...], wt_ref[...]) + btr_ref[...]
    h_ref[...] = h
    hws1_ref[...] = disc_ref[...] * _dot(h, w1_ref[...])


def _dis_body(degp_ref, disc_ref):
    ssum = jnp.sum(degp_ref[...], axis=0) + 1.0
    d = lax.rsqrt(ssum)
    disc_ref[...] = jnp.broadcast_to(d[:, None], (NP, H))


def _mid_body(sp_ref, hws1_ref, disc_ref, b1_ref, w2_ref, hws2_ref):
    dis = disc_ref[...]
    out1 = dis * (sp_ref[0] + sp_ref[1] + hws1_ref[...]) + b1_ref[...]
    out1 = jnp.maximum(out1, 0.0)
    hws2_ref[...] = dis * _dot(out1, w2_ref[...])


def _final_body(sp_ref, hws2_ref, disc_ref, b2_ref, bb_ref, y_ref):
    i = pl.program_id(0)
    dis = disc_ref[...]
    x1 = dis * (sp_ref[0] + sp_ref[1] + hws2_ref[...]) + b2_ref[...]
    bo = (bb_ref[...] == lax.broadcasted_iota(jnp.int32, (RB, G), 1)).astype(_F32)
    contrib = lax.dot_general(bo, x1, (((0,), (0,)), ((), ())),
                              precision=_HIGH, preferred_element_type=_F32)

    @pl.when(i == 0)
    def _():
        y_ref[...] = contrib

    @pl.when(i > 0)
    def _():
        y_ref[...] = y_ref[...] + contrib


def _prologue(xb, pe8, emb32, wt8, btr, w1, disc):
    return pl.pallas_call(
        _prologue_body,
        grid=(NBLK,),
        in_specs=[
            pl.BlockSpec((RB, 32), lambda i: (i, 0)),
            pl.BlockSpec((RB, 8), lambda i: (i, 0)),
            pl.BlockSpec((32, H), lambda i: (0, 0)),
            pl.BlockSpec((8, H), lambda i: (0, 0)),
            pl.BlockSpec((1, H), lambda i: (0, 0)),
            pl.BlockSpec((H, H), lambda i: (0, 0)),
            pl.BlockSpec((RB, H), lambda i: (i, 0)),
        ],
        out_specs=[
            pl.BlockSpec((RB, H), lambda i: (i, 0)),
            pl.BlockSpec((RB, H), lambda i: (i, 0)),
        ],
        out_shape=[
            jax.ShapeDtypeStruct((NP, H), _F32),
            jax.ShapeDtypeStruct((NP, H), _F32),
        ],
    )(xb, pe8, emb32, wt8, btr, w1, disc)


def _dis_kernel(degp):
    return pl.pallas_call(
        _dis_body,
        out_shape=jax.ShapeDtypeStruct((NP, H), _F32),
    )(degp)


def _mid(sp, hw1, disc, b1, w2):
    return pl.pallas_call(
        _mid_body,
        grid=(NBLK,),
        in_specs=[
            pl.BlockSpec((2, RB, H), lambda i: (0, i, 0)),
            pl.BlockSpec((RB, H), lambda i: (i, 0)),
            pl.BlockSpec((RB, H), lambda i: (i, 0)),
            pl.BlockSpec((1, H), lambda i: (0, 0)),
            pl.BlockSpec((H, H), lambda i: (0, 0)),
        ],
        out_specs=pl.BlockSpec((RB, H), lambda i: (i, 0)),
        out_shape=jax.ShapeDtypeStruct((NP, H), _F32),
    )(sp, hw1, disc, b1, w2)


def _final(sp, hw2, disc, b2, bb):
    return pl.pallas_call(
        _final_body,
        grid=(NBLK,),
        in_specs=[
            pl.BlockSpec((2, RB, H), lambda i: (0, i, 0)),
            pl.BlockSpec((RB, H), lambda i: (i, 0)),
            pl.BlockSpec((RB, H), lambda i: (i, 0)),
            pl.BlockSpec((1, H), lambda i: (0, 0)),
            pl.BlockSpec((RB, G), lambda i: (i, 0)),
        ],
        out_specs=pl.BlockSpec((G, H), lambda i: (0, 0)),
        out_shape=jax.ShapeDtypeStruct((G, H), _F32),
    )(sp, hw2, disc, b2, bb)


def kernel(x, laplacian_eigenvector_pe, edge_index, edge_attr, batch,
           embed_table, W_trans, b_trans, W1, b1, W2, b2):
    xb = jnp.broadcast_to(x[:, :1].astype(jnp.int32), (N, 32))
    xb = jnp.pad(xb, ((0, NP - N), (0, 0)))
    pe8 = jnp.pad(laplacian_eigenvector_pe, ((0, NP - N), (0, 3)))
    emb32 = jnp.pad(embed_table, ((0, 4), (0, 0)))
    wt8 = jnp.pad(W_trans, ((0, 3), (0, 0)))
    row_p = jnp.pad(edge_index[0].astype(jnp.int32), (0, EP - E))
    col_p = jnp.pad(edge_index[1].astype(jnp.int32), (0, EP - E))
    ew_p = jnp.pad(edge_attr, (0, EP - E))
    bb = jnp.pad(jnp.broadcast_to(batch[:, None].astype(jnp.int32), (N, G)),
                 ((0, NP - N), (0, 0)), constant_values=G)
    btr = b_trans[None, :]
    b1r = b1[None, :]
    b2r = b2[None, :]
    zeros2d = jnp.zeros((NP, H), _F32)

    row3 = row_p.reshape(TOT_CH, CH)
    col3 = col_p.reshape(TOT_CH, CH)
    ewb3 = lax.bitcast_convert_type(ew_p, jnp.int32).reshape(TOT_CH, CH)
    cw = jnp.stack([row3, col3, ewb3], axis=1)

    degp = _deg_kernel(col_p, ew_p)
    disc = _dis_kernel(degp)
    h_full, hws1 = _prologue(xb, pe8, emb32, wt8, btr, W1, disc)
    s1 = _msg_kernel(hws1, row3, cw, zeros2d)
    hws2 = _mid(s1, hws1, disc, b1r, W2)
    s2 = _msg_kernel(hws2, row3, cw, zeros2d)
    y = _final(s2, hws2, disc, b2r, bb)
    return (y, h_full[:N])


# speedup vs baseline: 1.1035x; 1.1035x over previous
import dataclasses
import functools

import jax
import jax.numpy as jnp
from jax import lax
from jax.experimental import pallas as pl
from jax.experimental.pallas import tpu as pltpu
from jax.experimental.pallas import tpu_sc as plsc

N = 10000
NP = 10240
E = 320000
EP = 327680
H = 128
G = 64
VOCAB = 28
RB = 1024
NBLK = NP // RB
NTILES = 32
E_TILE = EP // NTILES
CH = 80
N_CH = E_TILE // CH
DCH = 1024
N_DCH = E_TILE // DCH
ROWS_PER_TILE = NP // 16

_F32 = jnp.float32
_HIGH = lax.Precision.HIGHEST

_mesh = plsc.VectorSubcoreMesh(core_axis_name="c", subcore_axis_name="s",
                               num_cores=2, num_subcores=16)

_sc_params = pltpu.CompilerParams()
if "needs_layout_passes" in pltpu.CompilerParams.__dataclass_fields__:
    _sc_params = dataclasses.replace(_sc_params, needs_layout_passes=False)


@functools.partial(
    pl.kernel,
    out_type=jax.ShapeDtypeStruct((NTILES, NP), _F32),
    mesh=_mesh,
    scratch_types=[
        pltpu.VMEM((NP,), _F32),
        pltpu.VMEM((DCH,), jnp.int32),
        pltpu.VMEM((DCH,), _F32),
    ],
    compiler_params=_sc_params,
)
def _deg_kernel(col_hbm, ew_hbm, out_hbm, deg_ref, col_ref, ew_ref):
    c = lax.axis_index("c")
    s = lax.axis_index("s")
    tile = s * 2 + c
    zeros16 = jnp.zeros((16,), _F32)

    @pl.loop(0, NP // 16)
    def _(i):
        deg_ref[pl.ds(i * 16, 16)] = zeros16

    base = tile * E_TILE

    @pl.loop(0, N_DCH)
    def _(k):
        off = base + k * DCH
        pltpu.sync_copy(col_hbm.at[pl.ds(off, DCH)], col_ref)
        pltpu.sync_copy(ew_hbm.at[pl.ds(off, DCH)], ew_ref)

        @pl.loop(0, DCH // 16)
        def _(g):
            c16 = col_ref[pl.ds(g * 16, 16)]
            e16 = ew_ref[pl.ds(g * 16, 16)]
            plsc.addupdate_scatter(deg_ref, [c16], e16)

    pltpu.sync_copy(deg_ref, out_hbm.at[tile])


NB = 4
TOT_CH = EP // CH
NCH_C0 = 128
NCH_C1 = 128
assert NCH_C0 % NB == 0 and NCH_C1 % NB == 0
assert 16 * (NCH_C0 + NCH_C1) == TOT_CH

def _edge_pipeline(base, nch, hp_hbm, row_hbm, cw_hbm,
                   ROW, CW, M, SR, SCW, SG, SS, acc_ref):
    for b in range(min(NB, nch)):
        pltpu.async_copy(row_hbm.at[base + b], ROW[b], SR[b])
    for b in range(min(2, nch)):
        pltpu.async_copy(cw_hbm.at[base + b], CW[b], SCW[b])
    for b in range(min(2, nch)):
        pltpu.make_async_copy(row_hbm.at[base + b], ROW[b], SR[b]).wait()
        pltpu.async_copy(hp_hbm.at[ROW[b]], M[b], SG[b])

    @pl.loop(0, nch // NB)
    def _(kk):
        for b in range(NB):
            k = kk * NB + b
            b2 = (b + 2) % NB

            @pl.when(k >= 2)
            def _():
                pltpu.make_async_copy(
                    M[b2], acc_ref.at[CW[b2].at[0]], SS[b2]).wait()

            @pl.when(k + 2 < nch)
            def _():
                pltpu.async_copy(cw_hbm.at[base + k + 2], CW[b2], SCW[b2])
                pltpu.make_async_copy(
                    row_hbm.at[base + k + 2], ROW[b2], SR[b2]).wait()
                pltpu.async_copy(hp_hbm.at[ROW[b2]], M[b2], SG[b2])

            pltpu.make_async_copy(
                hp_hbm.at[ROW[b]], M[b], SG[b]).wait()

            @pl.when(k + 4 < nch)
            def _():
                pltpu.async_copy(row_hbm.at[base + k + 4], ROW[b], SR[b])

            pltpu.make_async_copy(cw_hbm.at[base + k], CW[b], SCW[b]).wait()

            @pl.loop(0, CH // 16)
            def _(g):
                e16 = plsc.bitcast(CW[b][1, pl.ds(g * 16, 16)], _F32)
                for j in range(16):
                    bj = jnp.broadcast_to(e16[j], (16,))
                    r = g * 16 + j
                    for f in range(8):
                        msl = (r, pl.ds(f * 16, 16))
                        M[b][msl] = M[b][msl] * bj

            pltpu.async_copy(M[b], acc_ref.at[CW[b].at[0]], SS[b], add=True)

    if nch >= 2:
        pltpu.make_async_copy(
            M[(nch - 2) % NB], acc_ref.at[CW[(nch - 2) % NB].at[0]],
            SS[(nch - 2) % NB]).wait()
    if nch >= 1:
        pltpu.make_async_copy(
            M[(nch - 1) % NB], acc_ref.at[CW[(nch - 1) % NB].at[0]],
            SS[(nch - 1) % NB]).wait()


@functools.partial(
    pl.kernel,
    out_type=jax.ShapeDtypeStruct((2, NP, H), _F32),
    mesh=_mesh,
    scratch_types=(
        [pltpu.VMEM((CH,), jnp.int32)] * NB
        + [pltpu.VMEM((2, CH), jnp.int32)] * NB
        + [pltpu.VMEM((CH, H), _F32)] * NB
        + [pltpu.VMEM_SHARED((NP, H), _F32)]
        + [pltpu.SemaphoreType.DMA] * (4 * NB)
    ),
    compiler_params=_sc_params,
)
def _msg_kernel(hp_hbm, row_hbm, cw_hbm, zeros_hbm, out_hbm,
                r0, r1, r2, r3, w0, w1, w2, w3, m0, m1, m2, m3,
                acc_ref, *sems):
    c = lax.axis_index("c")
    s = lax.axis_index("s")
    ROW = (r0, r1, r2, r3)
    CW = (w0, w1, w2, w3)
    M = (m0, m1, m2, m3)
    SR = sems[0:4]
    SCW = sems[4:8]
    SG = sems[8:12]
    SS = sems[12:16]
    pltpu.sync_copy(zeros_hbm.at[pl.ds(s * ROWS_PER_TILE, ROWS_PER_TILE)],
                    acc_ref.at[pl.ds(s * ROWS_PER_TILE, ROWS_PER_TILE)])
    plsc.subcore_barrier()

    if NCH_C0 > 0:
        @pl.when(c == 0)
        def _():
            _edge_pipeline(s * NCH_C0, NCH_C0, hp_hbm, row_hbm, cw_hbm,
                           ROW, CW, M, SR, SCW, SG, SS, acc_ref)
    if NCH_C1 > 0:
        @pl.when(c == 1)
        def _():
            _edge_pipeline(16 * NCH_C0 + s * NCH_C1, NCH_C1,
                           hp_hbm, row_hbm, cw_hbm,
                           ROW, CW, M, SR, SCW, SG, SS, acc_ref)

    plsc.subcore_barrier()
    pltpu.sync_copy(acc_ref.at[pl.ds(s * ROWS_PER_TILE, ROWS_PER_TILE)],
                    out_hbm.at[c].at[pl.ds(s * ROWS_PER_TILE, ROWS_PER_TILE)])


def _dot(a, b):
    return lax.dot_general(a, b, (((1,), (0,)), ((), ())),
                           precision=_HIGH, preferred_element_type=_F32)


def _prologue_body(xb_ref, pe_ref, emb_ref, wt_ref, btr_ref, w1_ref, disc_ref,
                   h_ref, hws1_ref):
    xb = xb_ref[...]
    iot = lax.broadcasted_iota(jnp.int32, (RB, 32), 1)
    oh = (xb == iot).astype(_F32)
    h = _dot(oh, emb_ref[...]) + _dot(pe_ref[...], wt_ref[...]) + btr_ref[...]
    h_ref[...] = h
    hws1_ref[...] = disc_ref[...] * _dot(h, w1_ref[...])


def _dis_body(degp_ref, disc_ref):
    ssum = jnp.sum(degp_ref[...], axis=0) + 1.0
    d = lax.rsqrt(ssum)
    disc_ref[...] = jnp.broadcast_to(d[:, None], (NP, H))


def _mid_body(sp_ref, hws1_ref, disc_ref, b1_ref, w2_ref, hws2_ref):
    dis = disc_ref[...]
    out1 = dis * (sp_ref[0] + sp_ref[1] + hws1_ref[...]) + b1_ref[...]
    out1 = jnp.maximum(out1, 0.0)
    hws2_ref[...] = dis * _dot(out1, w2_ref[...])


def _final_body(sp_ref, hws2_ref, disc_ref, b2_ref, bb_ref, y_ref):
    i = pl.program_id(0)
    dis = disc_ref[...]
    x1 = dis * (sp_ref[0] + sp_ref[1] + hws2_ref[...]) + b2_ref[...]
    bo = (bb_ref[...] == lax.broadcasted_iota(jnp.int32, (RB, G), 1)).astype(_F32)
    contrib = lax.dot_general(bo, x1, (((0,), (0,)), ((), ())),
                              precision=_HIGH, preferred_element_type=_F32)

    @pl.when(i == 0)
    def _():
        y_ref[...] = contrib

    @pl.when(i > 0)
    def _():
        y_ref[...] = y_ref[...] + contrib


def _prologue(xb, pe8, emb32, wt8, btr, w1, disc):
    return pl.pallas_call(
        _prologue_body,
        grid=(NBLK,),
        in_specs=[
            pl.BlockSpec((RB, 32), lambda i: (i, 0)),
            pl.BlockSpec((RB, 8), lambda i: (i, 0)),
            pl.BlockSpec((32, H), lambda i: (0, 0)),
            pl.BlockSpec((8, H), lambda i: (0, 0)),
            pl.BlockSpec((1, H), lambda i: (0, 0)),
            pl.BlockSpec((H, H), lambda i: (0, 0)),
            pl.BlockSpec((RB, H), lambda i: (i, 0)),
        ],
        out_specs=[
            pl.BlockSpec((RB, H), lambda i: (i, 0)),
            pl.BlockSpec((RB, H), lambda i: (i, 0)),
        ],
        out_shape=[
            jax.ShapeDtypeStruct((NP, H), _F32),
            jax.ShapeDtypeStruct((NP, H), _F32),
        ],
    )(xb, pe8, emb32, wt8, btr, w1, disc)


def _dis_kernel(degp):
    return pl.pallas_call(
        _dis_body,
        out_shape=jax.ShapeDtypeStruct((NP, H), _F32),
    )(degp)


def _mid(sp, hw1, disc, b1, w2):
    return pl.pallas_call(
        _mid_body,
        grid=(NBLK,),
        in_specs=[
            pl.BlockSpec((2, RB, H), lambda i: (0, i, 0)),
            pl.BlockSpec((RB, H), lambda i: (i, 0)),
            pl.BlockSpec((RB, H), lambda i: (i, 0)),
            pl.BlockSpec((1, H), lambda i: (0, 0)),
            pl.BlockSpec((H, H), lambda i: (0, 0)),
        ],
        out_specs=pl.BlockSpec((RB, H), lambda i: (i, 0)),
        out_shape=jax.ShapeDtypeStruct((NP, H), _F32),
    )(sp, hw1, disc, b1, w2)


def _final(sp, hw2, disc, b2, bb):
    return pl.pallas_call(
        _final_body,
        grid=(NBLK,),
        in_specs=[
            pl.BlockSpec((2, RB, H), lambda i: (0, i, 0)),
            pl.BlockSpec((RB, H), lambda i: (i, 0)),
            pl.BlockSpec((RB, H), lambda i: (i, 0)),
            pl.BlockSpec((1, H), lambda i: (0, 0)),
            pl.BlockSpec((RB, G), lambda i: (i, 0)),
        ],
        out_specs=pl.BlockSpec((G, H), lambda i: (0, 0)),
        out_shape=jax.ShapeDtypeStruct((G, H), _F32),
    )(sp, hw2, disc, b2, bb)


def kernel(x, laplacian_eigenvector_pe, edge_index, edge_attr, batch,
           embed_table, W_trans, b_trans, W1, b1, W2, b2):
    xb = jnp.broadcast_to(x[:, :1].astype(jnp.int32), (N, 32))
    xb = jnp.pad(xb, ((0, NP - N), (0, 0)))
    pe8 = jnp.pad(laplacian_eigenvector_pe, ((0, NP - N), (0, 3)))
    emb32 = jnp.pad(embed_table, ((0, 4), (0, 0)))
    wt8 = jnp.pad(W_trans, ((0, 3), (0, 0)))
    row_p = jnp.pad(edge_index[0].astype(jnp.int32), (0, EP - E))
    col_p = jnp.pad(edge_index[1].astype(jnp.int32), (0, EP - E))
    ew_p = jnp.pad(edge_attr, (0, EP - E))
    bb = jnp.pad(jnp.broadcast_to(batch[:, None].astype(jnp.int32), (N, G)),
                 ((0, NP - N), (0, 0)), constant_values=G)
    btr = b_trans[None, :]
    b1r = b1[None, :]
    b2r = b2[None, :]
    zeros2d = jnp.zeros((NP, H), _F32)

    row3 = row_p.reshape(TOT_CH, CH)
    col3 = col_p.reshape(TOT_CH, CH)
    ewb3 = lax.bitcast_convert_type(ew_p, jnp.int32).reshape(TOT_CH, CH)
    cw = jnp.stack([col3, ewb3], axis=1)

    degp = _deg_kernel(col_p, ew_p)
    disc = _dis_kernel(degp)
    h_full, hws1 = _prologue(xb, pe8, emb32, wt8, btr, W1, disc)
    s1 = _msg_kernel(hws1, row3, cw, zeros2d)
    hws2 = _mid(s1, hws1, disc, b1r, W2)
    s2 = _msg_kernel(hws2, row3, cw, zeros2d)
    y = _final(s2, hws2, disc, b2r, bb)
    return (y, h_full[:N])
